# baseline (speedup 1.0000x reference)
import numpy as np
import jax
import jax.numpy as jnp
from jax import lax
from jax.experimental import pallas as pl
from jax.experimental.pallas import tpu as pltpu
from jax.experimental.pallas import tpu_sc as plsc

_ROWS, _COLS = 64, 100000
_BLOCK_ROWS = 8
_SC_ROWS = 8
_TC_ROWS = _ROWS - _SC_ROWS
_NB = _TC_ROWS // _BLOCK_ROWS
_C = 1024
_NFULL = _COLS // _C
_TAIL = _COLS - _NFULL * _C

_K0 = np.uint32(0)
_K1 = np.uint32(42)
_KS2 = np.uint32(np.uint32(0x1BD11BDA) ^ _K0 ^ _K1)
_ROT = ((13, 15, 26, 6), (17, 29, 16, 24))
_SCHED = ((1, 2), (2, 0), (0, 1), (1, 2), (2, 0))
_TINY = np.float32(np.finfo(np.float32).tiny)


def _exp_gumbel(x1_init, load_x):
    ks = (_K0, _K1, _KS2)
    x1 = x1_init
    x0 = x1
    x1 = x0 ^ ((x1 << jnp.uint32(13)) | (x1 >> jnp.uint32(19)))
    first = True
    for r in range(5):
        for d in _ROT[r % 2]:
            if first:
                first = False
                continue
            x0 = x0 + x1
            x1 = (x1 << jnp.uint32(d)) | (x1 >> jnp.uint32(32 - d))
            x1 = x0 ^ x1
        a, b = _SCHED[r]
        if int(ks[a]):
            x0 = x0 + ks[a]
        x1 = x1 + np.uint32(ks[b] + np.uint32(r + 1))
    bits = x0 ^ x1

    fb = (bits >> jnp.uint32(9)) | jnp.uint32(0x3F800000)
    f = lax.bitcast_convert_type(fb, jnp.float32) - jnp.float32(1.0)
    u = f + _TINY
    L = jnp.float32(0.0) - jnp.log2(u)
    return jnp.exp2(load_x() * jnp.float32(np.log2(np.e))
                    - jnp.float32(np.log2(np.log(2.0)))) / L


def _tree_sum_128(e):
    parts = [e[:, k * 128:(k + 1) * 128] for k in range(e.shape[1] // 128)]
    while len(parts) > 1:
        parts = [parts[k] + parts[k + 1] for k in range(0, len(parts) - 1, 2)] \
            + ([parts[-1]] if len(parts) % 2 else [])
    return parts[0]


def _gumbel_softmax_block(x_ref, o_ref, e_ref, r_ref):
    i = pl.program_id(0)
    ph = lax.rem(i, 2)
    prev = 1 - ph
    r_prev = r_ref[prev, :, 0:1]

    @pl.when(i < _NB)
    def _gen():
        base = (i * (_BLOCK_ROWS * _COLS)).astype(jnp.uint32)
        row = lax.broadcasted_iota(jnp.uint32, (_BLOCK_ROWS, _C), 0)
        lane = lax.broadcasted_iota(jnp.uint32, (_BLOCK_ROWS, _C), 1)
        q = base + row * jnp.uint32(_COLS) + lane + _K1

        def gen_body(j, acc):
            start = j * _C
            e = _exp_gumbel(q + start.astype(jnp.uint32),
                            lambda: x_ref[:, pl.ds(start, _C)])
            e_ref[ph, :, pl.ds(start, _C)] = e
            o_ref[:, pl.ds(start, _C)] = e_ref[prev, :, pl.ds(start, _C)] * r_prev
            return acc + _tree_sum_128(e)

        acc = lax.fori_loop(0, _NFULL, gen_body,
                            jnp.zeros((_BLOCK_ROWS, 128), jnp.float32),
                            unroll=8)

        tail0 = _NFULL * _C
        row_t = lax.broadcasted_iota(jnp.uint32, (_BLOCK_ROWS, _TAIL), 0)
        lane_t = lax.broadcasted_iota(jnp.uint32, (_BLOCK_ROWS, _TAIL), 1)
        q_t = base + row_t * jnp.uint32(_COLS) + lane_t \
            + np.uint32(int(_K1) + tail0)
        e_t = _exp_gumbel(q_t, lambda: x_ref[:, pl.ds(tail0, _TAIL)])
        e_ref[ph, :, pl.ds(tail0, _TAIL)] = e_t
        o_ref[:, pl.ds(tail0, _TAIL)] = \
            e_ref[prev, :, pl.ds(tail0, _TAIL)] * r_prev

        s = jnp.sum(acc, axis=-1, keepdims=True) \
            + jnp.sum(e_t, axis=-1, keepdims=True)
        r_ref[ph] = jnp.broadcast_to(jnp.float32(1.0) / s, (_BLOCK_ROWS, 128))

    @pl.when(i == _NB)
    def _final_scale():
        o_ref[...] = e_ref[prev] * r_prev


_SCHUNK = 3200
_SC_NCH = _SCHUNK // 16
_SC_TSTART = 31 * _SCHUNK
_SC_TCH = _COLS - _SC_TSTART
_SC_TNCH = _SC_TCH // 16
_P5 = (-0.20438587444616144, 0.3147088056222172, -0.3659298827092522,
       0.4800737469156505, -0.7212366511576773, 1.4426991769054538)
_SQRT2 = np.float32(np.sqrt(2.0))
_LN2 = np.float32(np.log(2.0))


def _threefry_bits(x1_init):
    ks = (_K0, _K1, _KS2)
    x1 = x1_init
    x0 = x1
    x1 = x0 ^ ((x1 << jnp.uint32(13)) | (x1 >> jnp.uint32(19)))
    first = True
    for r in range(5):
        for d in _ROT[r % 2]:
            if first:
                first = False
                continue
            x0 = x0 + x1
            x1 = (x1 << jnp.uint32(d)) | (x1 >> jnp.uint32(32 - d))
            x1 = x0 ^ x1
        a, b = _SCHED[r]
        if int(ks[a]):
            x0 = x0 + ks[a]
        x1 = x1 + np.uint32(ks[b] + np.uint32(r + 1))
    return x0 ^ x1


def _sc_exp_gumbel(p_u32, x):
    bits = _threefry_bits(p_u32 + _K1)
    fb = (bits >> jnp.uint32(9)) | jnp.uint32(0x3F800000)
    f = lax.bitcast_convert_type(fb, jnp.float32) - jnp.float32(1.0)
    u = f + _TINY
    ub = lax.bitcast_convert_type(u, jnp.uint32)
    E = (ub >> jnp.uint32(23)).astype(jnp.int32) - 127
    m = lax.bitcast_convert_type(
        (ub & jnp.uint32(0x7FFFFF)) | jnp.uint32(0x3F800000), jnp.float32)
    big = m > _SQRT2
    m = jnp.where(big, m * jnp.float32(0.5), m)
    Ef = (E + jnp.where(big, 1, 0)).astype(jnp.float32)
    t = m - jnp.float32(1.0)
    pt = jnp.float32(_P5[0])
    for cc in _P5[1:]:
        pt = pt * t + jnp.float32(cc)
    L = jnp.float32(0.0) - (Ef + t * pt)
    return jnp.exp(x) / (L * _LN2)


def _sc_chunk(x_hbm, e_hbm, xb, eb, cstart, nch, ch, unroll):
    pltpu.sync_copy(x_hbm.at[pl.ds(56, 8), pl.ds(cstart, ch)], xb)
    lane_u = lax.iota(jnp.uint32, 16)
    for k in range(_SC_ROWS):
        pbase = (jnp.asarray((56 + k) * _COLS) + cstart).astype(jnp.uint32)

        def gen(j0, carry, k=k, pbase=pbase):
            for jj in range(unroll):
                j = j0 * unroll + jj
                x = xb[k, pl.ds(j * 16, 16)]
                p = pbase + jnp.uint32(16) * j.astype(jnp.uint32) + lane_u
                eb[k, pl.ds(j * 16, 16)] = _sc_exp_gumbel(p, x)
            return carry

        lax.fori_loop(0, nch // unroll, gen, jnp.int32(0))
    pltpu.sync_copy(eb, e_hbm.at[pl.ds(0, 8), pl.ds(cstart, ch)])


def _sc_rows_kernel(x_hbm, e_hbm, xb, eb, xbt, ebt):
    c = lax.axis_index("c")
    s = lax.axis_index("s")
    wid = c * 16 + s

    @pl.when(wid < 31)
    def _main():
        _sc_chunk(x_hbm, e_hbm, xb, eb, wid * _SCHUNK, _SC_NCH, _SCHUNK, 4)

    @pl.when(wid == 31)
    def _tail():
        _sc_chunk(x_hbm, e_hbm, xbt, ebt, _SC_TSTART, _SC_TNCH, _SC_TCH, 2)


def _sc_call(logits):
    import functools
    mesh = plsc.VectorSubcoreMesh(core_axis_name="c", subcore_axis_name="s")
    k = functools.partial(
        pl.kernel,
        mesh=mesh,
        out_type=jax.ShapeDtypeStruct((_SC_ROWS, _COLS), jnp.float32),
        scratch_types=[
            pltpu.VMEM((_SC_ROWS, _SCHUNK), jnp.float32),
            pltpu.VMEM((_SC_ROWS, _SCHUNK), jnp.float32),
            pltpu.VMEM((_SC_ROWS, _SC_TCH), jnp.float32),
            pltpu.VMEM((_SC_ROWS, _SC_TCH), jnp.float32),
        ],
    )(_sc_rows_kernel)
    return k(logits)


def _norm_block(tc_ref, e_ref, o_ref):
    del tc_ref
    s = jnp.sum(e_ref[...], axis=-1, keepdims=True)
    o_ref[...] = e_ref[...] * (jnp.float32(1.0) / s)


def _norm_call(tc, e):
    return pl.pallas_call(
        _norm_block,
        grid=(1,),
        in_specs=[
            pl.BlockSpec((_SC_ROWS, _COLS), lambda i: (_NB, 0)),
            pl.BlockSpec((_SC_ROWS, _COLS), lambda i: (0, 0)),
        ],
        out_specs=pl.BlockSpec((_SC_ROWS, _COLS), lambda i: (_NB, 0)),
        out_shape=jax.ShapeDtypeStruct((_ROWS, _COLS), jnp.float32),
        input_output_aliases={0: 0},
    )(tc, e)


def kernel(logits):
    sc_e = _sc_call(logits)
    tc = pl.pallas_call(
        _gumbel_softmax_block,
        grid=(_NB + 1,),
        in_specs=[pl.BlockSpec((_BLOCK_ROWS, _COLS),
                               lambda i: (jnp.minimum(i, _NB - 1), 0))],
        out_specs=pl.BlockSpec((_BLOCK_ROWS, _COLS),
                               lambda i: (jnp.maximum(i - 1, 0), 0)),
        out_shape=jax.ShapeDtypeStruct((_ROWS, _COLS), jnp.float32),
        scratch_shapes=[
            pltpu.VMEM((2, _BLOCK_ROWS, _COLS), jnp.float32),
            pltpu.VMEM((2, _BLOCK_ROWS, 128), jnp.float32),
        ],
        compiler_params=pltpu.CompilerParams(
            dimension_semantics=("arbitrary",),
        ),
    )(logits)

    return _norm_call(tc, sc_e)


# speedup vs baseline: 1.0726x; 1.0726x over previous
import numpy as np
import jax
import jax.numpy as jnp
from jax import lax
from jax.experimental import pallas as pl
from jax.experimental.pallas import tpu as pltpu

_ROWS, _COLS = 64, 100000
_BLOCK_ROWS = 8
_NB = _ROWS // _BLOCK_ROWS
_C = 1024
_NFULL = _COLS // _C
_TAIL = _COLS - _NFULL * _C

_K0 = np.uint32(0)
_K1 = np.uint32(42)
_KS2 = np.uint32(np.uint32(0x1BD11BDA) ^ _K0 ^ _K1)
_ROT = ((13, 15, 26, 6), (17, 29, 16, 24))
_SCHED = ((1, 2), (2, 0), (0, 1), (1, 2), (2, 0))
_TINY = np.float32(np.finfo(np.float32).tiny)


def _exp_gumbel(x1_init, load_x):
    ks = (_K0, _K1, _KS2)
    x1 = x1_init
    x0 = x1
    x1 = x0 ^ ((x1 << jnp.uint32(13)) | (x1 >> jnp.uint32(19)))
    first = True
    for r in range(5):
        for d in _ROT[r % 2]:
            if first:
                first = False
                continue
            x0 = x0 + x1
            x1 = (x1 << jnp.uint32(d)) | (x1 >> jnp.uint32(32 - d))
            x1 = x0 ^ x1
        a, b = _SCHED[r]
        if int(ks[a]):
            x0 = x0 + ks[a]
        x1 = x1 + np.uint32(ks[b] + np.uint32(r + 1))
    bits = x0 ^ x1

    fb = (bits >> jnp.uint32(9)) | jnp.uint32(0x3F800000)
    f = lax.bitcast_convert_type(fb, jnp.float32) - jnp.float32(1.0)
    u = f + _TINY
    L = jnp.float32(0.0) - jnp.log2(u)
    return jnp.exp2(load_x() * jnp.float32(np.log2(np.e))
                    - jnp.float32(np.log2(np.log(2.0)))) / L


def _tree_sum_128(e):
    parts = [e[:, k * 128:(k + 1) * 128] for k in range(e.shape[1] // 128)]
    while len(parts) > 1:
        parts = [parts[k] + parts[k + 1] for k in range(0, len(parts) - 1, 2)] \
            + ([parts[-1]] if len(parts) % 2 else [])
    return parts[0]


def _gumbel_softmax_block(x_ref, o_ref, e_ref, r_ref):
    i = pl.program_id(0)
    ph = lax.rem(i, 2)
    prev = 1 - ph
    r_prev = r_ref[prev, :, 0:1]

    @pl.when(i < _NB)
    def _gen():
        base = (i * (_BLOCK_ROWS * _COLS)).astype(jnp.uint32)
        row = lax.broadcasted_iota(jnp.uint32, (_BLOCK_ROWS, _C), 0)
        lane = lax.broadcasted_iota(jnp.uint32, (_BLOCK_ROWS, _C), 1)
        q = base + row * jnp.uint32(_COLS) + lane + _K1

        def gen_body(j, acc):
            start = j * _C
            e = _exp_gumbel(q + start.astype(jnp.uint32),
                            lambda: x_ref[:, pl.ds(start, _C)])
            e_ref[ph, :, pl.ds(start, _C)] = e
            o_ref[:, pl.ds(start, _C)] = e_ref[prev, :, pl.ds(start, _C)] * r_prev
            return acc + _tree_sum_128(e)

        acc = lax.fori_loop(0, _NFULL, gen_body,
                            jnp.zeros((_BLOCK_ROWS, 128), jnp.float32),
                            unroll=8)

        tail0 = _NFULL * _C
        row_t = lax.broadcasted_iota(jnp.uint32, (_BLOCK_ROWS, _TAIL), 0)
        lane_t = lax.broadcasted_iota(jnp.uint32, (_BLOCK_ROWS, _TAIL), 1)
        q_t = base + row_t * jnp.uint32(_COLS) + lane_t \
            + np.uint32(int(_K1) + tail0)
        e_t = _exp_gumbel(q_t, lambda: x_ref[:, pl.ds(tail0, _TAIL)])
        e_ref[ph, :, pl.ds(tail0, _TAIL)] = e_t
        o_ref[:, pl.ds(tail0, _TAIL)] = \
            e_ref[prev, :, pl.ds(tail0, _TAIL)] * r_prev

        s = jnp.sum(acc, axis=-1, keepdims=True) \
            + jnp.sum(e_t, axis=-1, keepdims=True)
        r_ref[ph] = jnp.broadcast_to(jnp.float32(1.0) / s, (_BLOCK_ROWS, 128))

    @pl.when(i == _NB)
    def _final_scale():
        o_ref[...] = e_ref[prev] * r_prev


def kernel(logits):
    return pl.pallas_call(
        _gumbel_softmax_block,
        grid=(_NB + 1,),
        in_specs=[pl.BlockSpec((_BLOCK_ROWS, _COLS),
                               lambda i: (jnp.minimum(i, _NB - 1), 0))],
        out_specs=pl.BlockSpec((_BLOCK_ROWS, _COLS),
                               lambda i: (jnp.maximum(i - 1, 0), 0)),
        out_shape=jax.ShapeDtypeStruct((_ROWS, _COLS), jnp.float32),
        scratch_shapes=[
            pltpu.VMEM((2, _BLOCK_ROWS, _COLS), jnp.float32),
            pltpu.VMEM((2, _BLOCK_ROWS, 128), jnp.float32),
        ],
        compiler_params=pltpu.CompilerParams(
            dimension_semantics=("arbitrary",),
        ),
    )(logits)
